# Initial kernel scaffold; baseline (speedup 1.0000x reference)
import jax
import jax.numpy as jnp
from jax.experimental import pallas as pl


def kernel(x, batch, W0_0, b0_0, g0_0, be0_0, W0_1, b0_1, g0_1, be0_1, W0_2, b0_2, g0_2, be0_2, W1_0, b1_0, g1_0, be1_0, W1_1, b1_1, g1_1, be1_1, W1_2, b1_2, g1_2, be1_2, W2_0, b2_0, g2_0, be2_0, W2_1, b2_1, g2_1, be2_1, W2_2, b2_2, g2_2, be2_2, Wfc1, bfc1, Wfc2, bfc2):
    raise NotImplementedError("write your pallas kernel here")


# speedup vs baseline: 8.1588x; 8.1588x over previous
import functools

import jax
import jax.numpy as jnp
from jax import lax
from jax.experimental import pallas as pl
from jax.experimental.pallas import tpu as pltpu
from jax.experimental.pallas import tpu_sc as plsc

N = 8192
NG = 64
K = 16
NK = N * K
RT = 128
CT = 256
NCT_MAX = N // CT
ET = RT * K
NT = N // RT
INF = float("inf")
BIG = 2 ** 30


def _topk_merge(rv, ri, cv, ci):
    allv = jnp.concatenate([rv, cv], axis=1)
    alli = jnp.concatenate([ri, ci], axis=1)
    outv, outi = [], []
    for _ in range(K):
        m = jnp.min(allv, axis=1, keepdims=True)
        eq = allv == m
        sel = jnp.min(jnp.where(eq, alli, BIG), axis=1, keepdims=True)
        hit = eq & (alli == sel)
        outv.append(m)
        outi.append(sel)
        allv = jnp.where(hit, INF, allv)
        alli = jnp.where(hit, BIG, alli)
    return jnp.concatenate(outv, axis=1), jnp.concatenate(outi, axis=1)


def _knn_body(pos_ref, batch_ref, out_ref, rv, ri, sm):
    r = pl.program_id(0)
    t = pl.program_id(1)
    R0 = pl.multiple_of(r * RT, RT)

    @pl.when(t == 0)
    def _():
        rb = batch_ref[pl.ds(R0, RT)]
        full = batch_ref[...]
        b0 = jnp.min(rb)
        b1 = jnp.max(rb)
        c0 = jnp.sum((full < b0).astype(jnp.int32))
        c1 = jnp.sum((full <= b1).astype(jnp.int32))
        c0a = (c0 // CT) * CT
        sm[0] = c0a
        sm[1] = (c1 - c0a + CT - 1) // CT
        colid = lax.broadcasted_iota(jnp.int32, (RT, CT), 1)
        rowid = R0 + lax.broadcasted_iota(jnp.int32, (RT, 1), 0)
        cb = batch_ref[pl.ds(0, CT)]
        elig = (cb[None, :] != rb[:, None]) | (colid == rowid)
        seedv = jnp.full((RT, CT), INF, jnp.float32)
        seedi = jnp.where(elig, colid, BIG)
        v0 = jnp.full((RT, K), INF, jnp.float32)
        i0 = jnp.full((RT, K), BIG, jnp.int32)
        nv, ni = _topk_merge(v0, i0, seedv, seedi)
        rv[...] = nv
        ri[...] = ni

    c0 = sm[0]
    nct = sm[1]

    @pl.when(t < nct)
    def _():
        rpos = pos_ref[pl.ds(R0, RT), :]
        rb = batch_ref[pl.ds(R0, RT)]
        rowid = R0 + lax.broadcasted_iota(jnp.int32, (RT, 1), 0)
        s = c0 + t * CT
        sc = pl.multiple_of(jnp.minimum(s, N - CT), CT)
        cpos = pos_ref[pl.ds(sc, CT), :]
        cb = batch_ref[pl.ds(sc, CT)]
        colid = sc + lax.broadcasted_iota(jnp.int32, (RT, CT), 1)
        p2r = jnp.sum(rpos * rpos, axis=1, keepdims=True)
        p2c = jnp.sum(cpos * cpos, axis=1)
        dot = lax.dot_general(rpos, cpos, (((1,), (1,)), ((), ())),
                              preferred_element_type=jnp.float32)
        d2 = p2r + p2c[None, :] - 2.0 * dot
        valid = ((cb[None, :] == rb[:, None]) & (colid != rowid)
                 & (colid >= s))
        cv = jnp.where(valid, d2, INF)
        ci = jnp.where(valid, colid, BIG)
        nv, ni = _topk_merge(rv[...], ri[...], cv, ci)
        rv[...] = nv
        ri[...] = ni

    @pl.when(t == NCT_MAX - 1)
    def _():
        out_ref[...] = ri[...].reshape(1, RT, K)


def _knn(pos, batch):
    c = pos.shape[1]
    out = pl.pallas_call(
        _knn_body,
        grid=(NT, NCT_MAX),
        in_specs=[
            pl.BlockSpec((N, c), lambda r, t: (0, 0)),
            pl.BlockSpec((N,), lambda r, t: (0,)),
        ],
        out_specs=pl.BlockSpec((1, RT, K), lambda r, t: (r, 0, 0)),
        out_shape=jax.ShapeDtypeStruct((NT, RT, K), jnp.int32),
        scratch_shapes=[
            pltpu.VMEM((RT, K), jnp.float32),
            pltpu.VMEM((RT, K), jnp.int32),
            pltpu.SMEM((4,), jnp.int32),
        ],
    )(pos, batch)
    return out.reshape(N, K)


def _ab_body(x_ref, wt_ref, wb_ref, b_ref, a_ref, bb_ref):
    xb = x_ref[...]
    wb = wb_ref[...]
    wdiff = wt_ref[...] - wb
    a_ref[...] = (lax.dot_general(xb, wdiff, (((1,), (0,)), ((), ())),
                                  preferred_element_type=jnp.float32)
                  + b_ref[...][None, :])
    bb_ref[...] = lax.dot_general(xb, wb, (((1,), (0,)), ((), ())),
                                  preferred_element_type=jnp.float32)


def _ab(x, w1, b1):
    c = x.shape[1]
    d = w1.shape[1]
    return pl.pallas_call(
        _ab_body,
        out_shape=(jax.ShapeDtypeStruct((N, d), jnp.float32),
                   jax.ShapeDtypeStruct((N, d), jnp.float32)),
    )(x, w1[:c], w1[c:], b1)


def _sc_gather(table, idx):
    d = table.shape[1]
    if d % 128 != 0:
        pad = 128 - d % 128
        return _sc_gather(jnp.pad(table, ((0, 0), (0, pad))), idx)[:, :d]
    nw = 32
    b_per_w = NK // nw
    ch = 128
    iters = b_per_w // ch
    mesh = plsc.VectorSubcoreMesh(core_axis_name="c", subcore_axis_name="s")

    @functools.partial(
        pl.kernel, mesh=mesh,
        out_type=jax.ShapeDtypeStruct((NK, d), jnp.float32),
        scratch_types=[
            pltpu.VMEM((ch,), jnp.int32),
            pltpu.VMEM((ch, d), jnp.float32),
            pltpu.SemaphoreType.DMA,
        ],
    )
    def gk(table_hbm, idx_hbm, out_hbm, idx_v, rows_v, sem):
        wid = lax.axis_index("s") * 2 + lax.axis_index("c")
        base = wid * b_per_w

        def body(ci, carry):
            off = base + ci * ch
            pltpu.sync_copy(idx_hbm.at[pl.ds(off, ch)], idx_v)
            pltpu.async_copy(table_hbm.at[idx_v], rows_v, sem).wait()
            pltpu.sync_copy(rows_v, out_hbm.at[pl.ds(off, ch)])
            return carry

        lax.fori_loop(0, iters, body, 0)

    return gk(table, idx)


def _bcast16(a, d):
    return jnp.broadcast_to(a[:, None, :], (RT, K, d)).reshape(ET, d)


def _bn_coeffs(stats_ref, g_ref, be_ref):
    mu = stats_ref[0, :] * (1.0 / NK)
    ex2 = stats_ref[1, :] * (1.0 / NK)
    var = ex2 - mu * mu
    rstd = lax.rsqrt(var + 1e-5)
    scale = g_ref[...] * rstd
    shift = be_ref[...] - mu * scale
    return scale, shift


def _acc_stats(stats_ref, h, r):
    @pl.when(r == 0)
    def _():
        stats_ref[...] = jnp.zeros_like(stats_ref)
    stats_ref[0, :] += jnp.sum(h, axis=0)
    stats_ref[1, :] += jnp.sum(h * h, axis=0)


def _p1_body(a_ref, bj_ref, stats_ref):
    r = pl.program_id(0)
    d = a_ref.shape[1]
    h = _bcast16(a_ref[...], d) + bj_ref[...]
    _acc_stats(stats_ref, h, r)


def _p2_body(a_ref, bj_ref, st1_ref, g_ref, be_ref, w_ref, b_ref,
             e_ref, st2_ref):
    r = pl.program_id(0)
    d = a_ref.shape[1]
    scale, shift = _bn_coeffs(st1_ref, g_ref, be_ref)
    h1 = _bcast16(a_ref[...], d) + bj_ref[...]
    h1 = jnp.maximum(h1 * scale[None, :] + shift[None, :], 0.0)
    e = (lax.dot_general(h1, w_ref[...], (((1,), (0,)), ((), ())),
                         preferred_element_type=jnp.float32)
         + b_ref[...][None, :])
    e_ref[...] = e
    _acc_stats(st2_ref, e, r)


def _p3_body(e1_ref, st2_ref, g_ref, be_ref, w_ref, b_ref, e2_ref, st3_ref):
    r = pl.program_id(0)
    scale, shift = _bn_coeffs(st2_ref, g_ref, be_ref)
    h2 = jnp.maximum(e1_ref[...] * scale[None, :] + shift[None, :], 0.0)
    e = (lax.dot_general(h2, w_ref[...], (((1,), (0,)), ((), ())),
                         preferred_element_type=jnp.float32)
         + b_ref[...][None, :])
    e2_ref[...] = e
    _acc_stats(st3_ref, e, r)


def _p4_body(e2_ref, st3_ref, g_ref, be_ref, h_ref):
    scale, shift = _bn_coeffs(st3_ref, g_ref, be_ref)
    d = e2_ref.shape[1]
    h3 = jnp.maximum(e2_ref[...] * scale[None, :] + shift[None, :], 0.0)
    h_ref[...] = jnp.mean(h3.reshape(RT, K, d), axis=1)


def _edge_block(x, idx, w0, b0, g0, be0, w1, b1, g1, be1, w2, b2, g2, be2):
    d = w0.shape[1]
    a, b = _ab(x, w0, b0)
    bj = _sc_gather(b, idx.reshape(NK))

    espec = pl.BlockSpec((ET, d), lambda r: (r, 0))
    aspec = pl.BlockSpec((RT, d), lambda r: (r, 0))
    sspec = pl.BlockSpec((8, d), lambda r: (0, 0))
    vspec = pl.BlockSpec((d,), lambda r: (0,))
    wspec = pl.BlockSpec((d, d), lambda r: (0, 0))
    sshape = jax.ShapeDtypeStruct((8, d), jnp.float32)

    st1 = pl.pallas_call(
        _p1_body, grid=(NT,),
        in_specs=[aspec, espec],
        out_specs=sspec, out_shape=sshape,
    )(a, bj)

    e1, st2 = pl.pallas_call(
        _p2_body, grid=(NT,),
        in_specs=[aspec, espec, sspec, vspec, vspec, wspec, vspec],
        out_specs=(espec, sspec),
        out_shape=(jax.ShapeDtypeStruct((NK, d), jnp.float32), sshape),
    )(a, bj, st1, g0, be0, w1, b1)

    e2, st3 = pl.pallas_call(
        _p3_body, grid=(NT,),
        in_specs=[espec, sspec, vspec, vspec, wspec, vspec],
        out_specs=(espec, sspec),
        out_shape=(jax.ShapeDtypeStruct((NK, d), jnp.float32), sshape),
    )(e1, st2, g1, be1, w2, b2)

    h = pl.pallas_call(
        _p4_body, grid=(NT,),
        in_specs=[espec, sspec, vspec, vspec],
        out_specs=aspec,
        out_shape=jax.ShapeDtypeStruct((N, d), jnp.float32),
    )(e2, st3, g2, be2)
    return h


def _pool_body(x_ref, b_ref, w1_ref, b1_ref, w2_ref, b2_ref, out_ref,
               acc, cnt):
    r = pl.program_id(0)
    xb = x_ref[...]
    bb = b_ref[pl.ds(r * RT, RT)]
    ind = (bb[:, None] == lax.broadcasted_iota(jnp.int32, (RT, NG), 1)
           ).astype(jnp.float32)

    @pl.when(r == 0)
    def _():
        acc[...] = jnp.zeros_like(acc)
        cnt[...] = jnp.zeros_like(cnt)

    acc[...] += lax.dot_general(ind, xb, (((0,), (0,)), ((), ())),
                                preferred_element_type=jnp.float32)
    cnt[...] += jnp.sum(ind, axis=0)

    @pl.when(r == NT - 1)
    def _():
        pooled = acc[...] / jnp.maximum(cnt[...], 1.0)[:, None]
        h = (lax.dot_general(pooled, w1_ref[...], (((1,), (0,)), ((), ())),
                             preferred_element_type=jnp.float32)
             + b1_ref[...][None, :])
        h = jnp.maximum(h, 0.0)
        out_ref[...] = (lax.dot_general(h, w2_ref[...],
                                        (((1,), (0,)), ((), ())),
                                        preferred_element_type=jnp.float32)
                        + b2_ref[...][None, :])


def _pool_fc(x3, batch, wfc1, bfc1, wfc2, bfc2):
    c = x3.shape[1]
    return pl.pallas_call(
        _pool_body,
        grid=(NT,),
        in_specs=[
            pl.BlockSpec((RT, c), lambda r: (r, 0)),
            pl.BlockSpec((N,), lambda r: (0,)),
            pl.BlockSpec((c, 256), lambda r: (0, 0)),
            pl.BlockSpec((256,), lambda r: (0,)),
            pl.BlockSpec((256, 5), lambda r: (0, 0)),
            pl.BlockSpec((5,), lambda r: (0,)),
        ],
        out_specs=pl.BlockSpec((NG, 5), lambda r: (0, 0)),
        out_shape=jax.ShapeDtypeStruct((NG, 5), jnp.float32),
        scratch_shapes=[
            pltpu.VMEM((NG, c), jnp.float32),
            pltpu.VMEM((NG,), jnp.float32),
        ],
    )(x3, batch, wfc1, bfc1, wfc2, bfc2)


def kernel(x, batch,
           W0_0, b0_0, g0_0, be0_0,
           W0_1, b0_1, g0_1, be0_1,
           W0_2, b0_2, g0_2, be0_2,
           W1_0, b1_0, g1_0, be1_0,
           W1_1, b1_1, g1_1, be1_1,
           W1_2, b1_2, g1_2, be1_2,
           W2_0, b2_0, g2_0, be2_0,
           W2_1, b2_1, g2_1, be2_1,
           W2_2, b2_2, g2_2, be2_2,
           Wfc1, bfc1, Wfc2, bfc2):
    batch = batch.astype(jnp.int32)
    params = [
        (W0_0, b0_0, g0_0, be0_0, W0_1, b0_1, g0_1, be0_1,
         W0_2, b0_2, g0_2, be0_2),
        (W1_0, b1_0, g1_0, be1_0, W1_1, b1_1, g1_1, be1_1,
         W1_2, b1_2, g1_2, be1_2),
        (W2_0, b2_0, g2_0, be2_0, W2_1, b2_1, g2_1, be2_1,
         W2_2, b2_2, g2_2, be2_2),
    ]
    for i in range(3):
        pos = x[:, :2] if i == 0 else x
        idx = _knn(pos, batch)
        h = _edge_block(x, idx, *params[i])
        x = jnp.concatenate([h, x], axis=1)
    return _pool_fc(x, batch, Wfc1, bfc1, Wfc2, bfc2)


# speedup vs baseline: 15.8936x; 1.9480x over previous
import functools

import jax
import jax.numpy as jnp
from jax import lax
from jax.experimental import pallas as pl
from jax.experimental.pallas import tpu as pltpu
from jax.experimental.pallas import tpu_sc as plsc

N = 8192
NG = 64
K = 16
NK = N * K
RT = 128
CT = 256
NCT_MAX = N // CT
ET = RT * K
NT = N // RT
INF = float("inf")
BIG = 2 ** 30


def _topk_merge(rv, ri, cv, ci):
    allv = jnp.concatenate([rv, cv], axis=0)
    alli = jnp.concatenate([ri, ci], axis=0)
    outv, outi = [], []
    for _ in range(K):
        m = jnp.min(allv, axis=0, keepdims=True)
        eq = allv == m
        sel = jnp.min(jnp.where(eq, alli, BIG), axis=0, keepdims=True)
        hit = eq & (alli == sel)
        outv.append(m)
        outi.append(sel)
        allv = jnp.where(hit, INF, allv)
        alli = jnp.where(hit, BIG, alli)
    return jnp.concatenate(outv, axis=0), jnp.concatenate(outi, axis=0)


def _knn_body(seed_ref, pos_ref, batch_ref, out_ref, rv, ri):
    r = pl.program_id(0)
    R0 = pl.multiple_of(r * RT, RT)
    rb = batch_ref[pl.ds(R0, RT)]
    full = batch_ref[...]
    b0 = jnp.min(rb)
    b1 = jnp.max(rb)
    c0 = jnp.sum((full < b0).astype(jnp.int32))
    c1 = jnp.sum((full <= b1).astype(jnp.int32))
    c0a = (c0 // CT) * CT
    nct = (c1 - c0a + CT - 1) // CT
    rowid = R0 + lax.broadcasted_iota(jnp.int32, (1, RT), 1)

    rv[...] = jnp.full((K, RT), INF, jnp.float32)
    ri[...] = jnp.full((K, RT), BIG, jnp.int32)

    @pl.when(seed_ref[r] != 0)
    def _():
        colid = lax.broadcasted_iota(jnp.int32, (CT, RT), 0)
        cb = batch_ref[pl.ds(0, CT)]
        elig = (cb[:, None] != rb[None, :]) | (colid == rowid)
        seedv = jnp.full((CT, RT), INF, jnp.float32)
        seedi = jnp.where(elig, colid, BIG)
        nv, ni = _topk_merge(rv[...], ri[...], seedv, seedi)
        rv[...] = nv
        ri[...] = ni

    rpos = pos_ref[pl.ds(R0, RT), :]
    p2r = jnp.sum(rpos * rpos, axis=1)[None, :]

    def tile_step(t, carry):
        s = c0a + t * CT
        sc = pl.multiple_of(jnp.minimum(s, N - CT), CT)
        cpos = pos_ref[pl.ds(sc, CT), :]
        cb = batch_ref[pl.ds(sc, CT)]
        colid = sc + lax.broadcasted_iota(jnp.int32, (CT, RT), 0)
        p2c = jnp.sum(cpos * cpos, axis=1)[:, None]
        dot = lax.dot_general(cpos, rpos, (((1,), (1,)), ((), ())),
                              preferred_element_type=jnp.float32)
        d2 = p2c + p2r - 2.0 * dot
        valid = ((cb[:, None] == rb[None, :]) & (colid != rowid)
                 & (colid >= s))
        cv = jnp.where(valid, d2, INF)
        ci = jnp.where(valid, colid, BIG)
        nv, ni = _topk_merge(rv[...], ri[...], cv, ci)
        rv[...] = nv
        ri[...] = ni
        return carry

    lax.fori_loop(0, nct, tile_step, 0)
    out_ref[...] = ri[...].reshape(1, K, RT)


def _knn(pos, batch, seed_flags):
    c = pos.shape[1]
    out = pl.pallas_call(
        _knn_body,
        grid=(NT,),
        in_specs=[
            pl.BlockSpec(memory_space=pltpu.SMEM),
            pl.BlockSpec((N, c), lambda r: (0, 0)),
            pl.BlockSpec((N,), lambda r: (0,)),
        ],
        out_specs=pl.BlockSpec((1, K, RT), lambda r: (r, 0, 0)),
        out_shape=jax.ShapeDtypeStruct((NT, K, RT), jnp.int32),
        scratch_shapes=[
            pltpu.VMEM((K, RT), jnp.float32),
            pltpu.VMEM((K, RT), jnp.int32),
        ],
    )(seed_flags, pos, batch)
    return out.transpose(0, 2, 1).reshape(N, K)


def _ab_body(x_ref, wt_ref, wb_ref, b_ref, a_ref, bb_ref):
    xb = x_ref[...]
    wb = wb_ref[...]
    wdiff = wt_ref[...] - wb
    a_ref[...] = (lax.dot_general(xb, wdiff, (((1,), (0,)), ((), ())),
                                  preferred_element_type=jnp.float32)
                  + b_ref[...][None, :])
    bb_ref[...] = lax.dot_general(xb, wb, (((1,), (0,)), ((), ())),
                                  preferred_element_type=jnp.float32)


def _ab(x, w1, b1):
    c = x.shape[1]
    d = w1.shape[1]
    return pl.pallas_call(
        _ab_body,
        out_shape=(jax.ShapeDtypeStruct((N, d), jnp.float32),
                   jax.ShapeDtypeStruct((N, d), jnp.float32)),
    )(x, w1[:c], w1[c:], b1)


def _sc_gather(table, idx):
    d = table.shape[1]
    if d % 128 != 0:
        pad = 128 - d % 128
        return _sc_gather(jnp.pad(table, ((0, 0), (0, pad))), idx)[:, :d]
    nw = 32
    b_per_w = NK // nw
    ch = 128
    iters = b_per_w // ch
    mesh = plsc.VectorSubcoreMesh(core_axis_name="c", subcore_axis_name="s")

    @functools.partial(
        pl.kernel, mesh=mesh,
        out_type=jax.ShapeDtypeStruct((NK, d), jnp.float32),
        scratch_types=[
            pltpu.VMEM((ch,), jnp.int32),
            pltpu.VMEM((ch, d), jnp.float32),
            pltpu.SemaphoreType.DMA,
        ],
    )
    def gk(table_hbm, idx_hbm, out_hbm, idx_v, rows_v, sem):
        wid = lax.axis_index("s") * 2 + lax.axis_index("c")
        base = wid * b_per_w

        def body(ci, carry):
            off = base + ci * ch
            pltpu.sync_copy(idx_hbm.at[pl.ds(off, ch)], idx_v)
            pltpu.async_copy(table_hbm.at[idx_v], rows_v, sem).wait()
            pltpu.sync_copy(rows_v, out_hbm.at[pl.ds(off, ch)])
            return carry

        lax.fori_loop(0, iters, body, 0)

    return gk(table, idx)


def _bcast16(a, d):
    return jnp.broadcast_to(a[:, None, :], (RT, K, d)).reshape(ET, d)


def _bn_coeffs(stats_ref, g_ref, be_ref):
    mu = stats_ref[0, :] * (1.0 / NK)
    ex2 = stats_ref[1, :] * (1.0 / NK)
    var = ex2 - mu * mu
    rstd = lax.rsqrt(var + 1e-5)
    scale = g_ref[...] * rstd
    shift = be_ref[...] - mu * scale
    return scale, shift


def _acc_stats(stats_ref, h, r):
    @pl.when(r == 0)
    def _():
        stats_ref[...] = jnp.zeros_like(stats_ref)
    stats_ref[0, :] += jnp.sum(h, axis=0)
    stats_ref[1, :] += jnp.sum(h * h, axis=0)


def _p1_body(a_ref, bj_ref, stats_ref):
    r = pl.program_id(0)
    d = a_ref.shape[1]
    h = _bcast16(a_ref[...], d) + bj_ref[...]
    _acc_stats(stats_ref, h, r)


def _p2_body(a_ref, bj_ref, st1_ref, g_ref, be_ref, w_ref, b_ref,
             e_ref, st2_ref):
    r = pl.program_id(0)
    d = a_ref.shape[1]
    scale, shift = _bn_coeffs(st1_ref, g_ref, be_ref)
    h1 = _bcast16(a_ref[...], d) + bj_ref[...]
    h1 = jnp.maximum(h1 * scale[None, :] + shift[None, :], 0.0)
    e = (lax.dot_general(h1, w_ref[...], (((1,), (0,)), ((), ())),
                         preferred_element_type=jnp.float32)
         + b_ref[...][None, :])
    e_ref[...] = e
    _acc_stats(st2_ref, e, r)


def _p3_body(e1_ref, st2_ref, g_ref, be_ref, w_ref, b_ref, e2_ref, st3_ref):
    r = pl.program_id(0)
    scale, shift = _bn_coeffs(st2_ref, g_ref, be_ref)
    h2 = jnp.maximum(e1_ref[...] * scale[None, :] + shift[None, :], 0.0)
    e = (lax.dot_general(h2, w_ref[...], (((1,), (0,)), ((), ())),
                         preferred_element_type=jnp.float32)
         + b_ref[...][None, :])
    e2_ref[...] = e
    _acc_stats(st3_ref, e, r)


def _p4_body(e2_ref, st3_ref, g_ref, be_ref, h_ref):
    scale, shift = _bn_coeffs(st3_ref, g_ref, be_ref)
    d = e2_ref.shape[1]
    h3 = jnp.maximum(e2_ref[...] * scale[None, :] + shift[None, :], 0.0)
    h_ref[...] = jnp.mean(h3.reshape(RT, K, d), axis=1)


def _edge_block(x, idx, w0, b0, g0, be0, w1, b1, g1, be1, w2, b2, g2, be2):
    d = w0.shape[1]
    a, b = _ab(x, w0, b0)
    bj = _sc_gather(b, idx.reshape(NK))

    espec = pl.BlockSpec((ET, d), lambda r: (r, 0))
    aspec = pl.BlockSpec((RT, d), lambda r: (r, 0))
    sspec = pl.BlockSpec((8, d), lambda r: (0, 0))
    vspec = pl.BlockSpec((d,), lambda r: (0,))
    wspec = pl.BlockSpec((d, d), lambda r: (0, 0))
    sshape = jax.ShapeDtypeStruct((8, d), jnp.float32)

    st1 = pl.pallas_call(
        _p1_body, grid=(NT,),
        in_specs=[aspec, espec],
        out_specs=sspec, out_shape=sshape,
    )(a, bj)

    e1, st2 = pl.pallas_call(
        _p2_body, grid=(NT,),
        in_specs=[aspec, espec, sspec, vspec, vspec, wspec, vspec],
        out_specs=(espec, sspec),
        out_shape=(jax.ShapeDtypeStruct((NK, d), jnp.float32), sshape),
    )(a, bj, st1, g0, be0, w1, b1)

    e2, st3 = pl.pallas_call(
        _p3_body, grid=(NT,),
        in_specs=[espec, sspec, vspec, vspec, wspec, vspec],
        out_specs=(espec, sspec),
        out_shape=(jax.ShapeDtypeStruct((NK, d), jnp.float32), sshape),
    )(e1, st2, g1, be1, w2, b2)

    h = pl.pallas_call(
        _p4_body, grid=(NT,),
        in_specs=[espec, sspec, vspec, vspec],
        out_specs=aspec,
        out_shape=jax.ShapeDtypeStruct((N, d), jnp.float32),
    )(e2, st3, g2, be2)
    return h


def _pool_body(x_ref, b_ref, w1_ref, b1_ref, w2_ref, b2_ref, out_ref,
               acc, cnt):
    r = pl.program_id(0)
    xb = x_ref[...]
    bb = b_ref[pl.ds(r * RT, RT)]
    ind = (bb[:, None] == lax.broadcasted_iota(jnp.int32, (RT, NG), 1)
           ).astype(jnp.float32)

    @pl.when(r == 0)
    def _():
        acc[...] = jnp.zeros_like(acc)
        cnt[...] = jnp.zeros_like(cnt)

    acc[...] += lax.dot_general(ind, xb, (((0,), (0,)), ((), ())),
                                preferred_element_type=jnp.float32)
    cnt[...] += jnp.sum(ind, axis=0)

    @pl.when(r == NT - 1)
    def _():
        pooled = acc[...] / jnp.maximum(cnt[...], 1.0)[:, None]
        h = (lax.dot_general(pooled, w1_ref[...], (((1,), (0,)), ((), ())),
                             preferred_element_type=jnp.float32)
             + b1_ref[...][None, :])
        h = jnp.maximum(h, 0.0)
        out_ref[...] = (lax.dot_general(h, w2_ref[...],
                                        (((1,), (0,)), ((), ())),
                                        preferred_element_type=jnp.float32)
                        + b2_ref[...][None, :])


def _pool_fc(x3, batch, wfc1, bfc1, wfc2, bfc2):
    c = x3.shape[1]
    return pl.pallas_call(
        _pool_body,
        grid=(NT,),
        in_specs=[
            pl.BlockSpec((RT, c), lambda r: (r, 0)),
            pl.BlockSpec((N,), lambda r: (0,)),
            pl.BlockSpec((c, 256), lambda r: (0, 0)),
            pl.BlockSpec((256,), lambda r: (0,)),
            pl.BlockSpec((256, 5), lambda r: (0, 0)),
            pl.BlockSpec((5,), lambda r: (0,)),
        ],
        out_specs=pl.BlockSpec((NG, 5), lambda r: (0, 0)),
        out_shape=jax.ShapeDtypeStruct((NG, 5), jnp.float32),
        scratch_shapes=[
            pltpu.VMEM((NG, c), jnp.float32),
            pltpu.VMEM((NG,), jnp.float32),
        ],
    )(x3, batch, wfc1, bfc1, wfc2, bfc2)


def kernel(x, batch,
           W0_0, b0_0, g0_0, be0_0,
           W0_1, b0_1, g0_1, be0_1,
           W0_2, b0_2, g0_2, be0_2,
           W1_0, b1_0, g1_0, be1_0,
           W1_1, b1_1, g1_1, be1_1,
           W1_2, b1_2, g1_2, be1_2,
           W2_0, b2_0, g2_0, be2_0,
           W2_1, b2_1, g2_1, be2_1,
           W2_2, b2_2, g2_2, be2_2,
           Wfc1, bfc1, Wfc2, bfc2):
    batch = batch.astype(jnp.int32)
    counts = jnp.zeros((NG,), jnp.int32).at[batch].add(1)
    b2d = batch.reshape(NT, RT)
    jets = jnp.arange(NG, dtype=jnp.int32)
    present = (jets[None, :] >= b2d[:, :1]) & (jets[None, :] <= b2d[:, -1:])
    minc = jnp.min(jnp.where(present, counts[None, :], BIG), axis=1)
    seed_flags = (minc <= K).astype(jnp.int32)
    params = [
        (W0_0, b0_0, g0_0, be0_0, W0_1, b0_1, g0_1, be0_1,
         W0_2, b0_2, g0_2, be0_2),
        (W1_0, b1_0, g1_0, be1_0, W1_1, b1_1, g1_1, be1_1,
         W1_2, b1_2, g1_2, be1_2),
        (W2_0, b2_0, g2_0, be2_0, W2_1, b2_1, g2_1, be2_1,
         W2_2, b2_2, g2_2, be2_2),
    ]
    for i in range(3):
        pos = x[:, :2] if i == 0 else x
        idx = _knn(pos, batch, seed_flags)
        h = _edge_block(x, idx, *params[i])
        x = jnp.concatenate([h, x], axis=1)
    return _pool_fc(x, batch, Wfc1, bfc1, Wfc2, bfc2)


# speedup vs baseline: 16.9564x; 1.0669x over previous
import functools

import jax
import jax.numpy as jnp
from jax import lax
from jax.experimental import pallas as pl
from jax.experimental.pallas import tpu as pltpu
from jax.experimental.pallas import tpu_sc as plsc

N = 8192
NG = 64
K = 16
NK = N * K
RT = 128
CT = 256
NCT_MAX = N // CT
ET = RT * K
NT = N // RT
INF = float("inf")
BIG = 2 ** 30


def _topk_merge(rv, ri, cv, ci):
    allv = jnp.concatenate([rv, cv], axis=0)
    alli = jnp.concatenate([ri, ci], axis=0)
    outv, outi = [], []
    for _ in range(K):
        m = jnp.min(allv, axis=0, keepdims=True)
        eq = allv == m
        sel = jnp.min(jnp.where(eq, alli, BIG), axis=0, keepdims=True)
        hit = eq & (alli == sel)
        outv.append(m)
        outi.append(sel)
        allv = jnp.where(hit, INF, allv)
        alli = jnp.where(hit, BIG, alli)
    return jnp.concatenate(outv, axis=0), jnp.concatenate(outi, axis=0)


def _knn_body(seed_ref, pos_ref, batch_ref, out_ref, rv, ri):
    r = pl.program_id(0)
    R0 = pl.multiple_of(r * RT, RT)
    rb = batch_ref[pl.ds(R0, RT)]
    full = batch_ref[...]
    b0 = jnp.min(rb)
    b1 = jnp.max(rb)
    c0 = jnp.sum((full < b0).astype(jnp.int32))
    c1 = jnp.sum((full <= b1).astype(jnp.int32))
    c0a = (c0 // 128) * 128
    nct = (c1 - c0a + CT - 1) // CT
    rowid = R0 + lax.broadcasted_iota(jnp.int32, (1, RT), 1)

    rv[...] = jnp.full((K, RT), INF, jnp.float32)
    ri[...] = jnp.full((K, RT), BIG, jnp.int32)

    @pl.when(seed_ref[r] != 0)
    def _():
        colid = lax.broadcasted_iota(jnp.int32, (CT, RT), 0)
        cb = batch_ref[pl.ds(0, CT)]
        elig = (cb[:, None] != rb[None, :]) | (colid == rowid)
        seedv = jnp.full((CT, RT), INF, jnp.float32)
        seedi = jnp.where(elig, colid, BIG)
        nv, ni = _topk_merge(rv[...], ri[...], seedv, seedi)
        rv[...] = nv
        ri[...] = ni

    rpos = pos_ref[pl.ds(R0, RT), :]
    p2r = jnp.sum(rpos * rpos, axis=1)[None, :]

    def tile_step(t, carry):
        s = c0a + t * CT
        sc = pl.multiple_of(jnp.minimum(s, N - CT), 128)
        cpos = pos_ref[pl.ds(sc, CT), :]
        cb = batch_ref[pl.ds(sc, CT)]
        colid = sc + lax.broadcasted_iota(jnp.int32, (CT, RT), 0)
        p2c = jnp.sum(cpos * cpos, axis=1)[:, None]
        dot = lax.dot_general(cpos, rpos, (((1,), (1,)), ((), ())),
                              preferred_element_type=jnp.float32)
        d2 = p2c + p2r - 2.0 * dot
        valid = ((cb[:, None] == rb[None, :]) & (colid != rowid)
                 & (colid >= s))
        cv = jnp.where(valid, d2, INF)
        ci = jnp.where(valid, colid, BIG)
        nv, ni = _topk_merge(rv[...], ri[...], cv, ci)
        rv[...] = nv
        ri[...] = ni
        return carry

    lax.fori_loop(0, nct, tile_step, 0)
    out_ref[...] = ri[...].reshape(1, K, RT)


def _knn(pos, batch, seed_flags):
    c = pos.shape[1]
    out = pl.pallas_call(
        _knn_body,
        grid=(NT,),
        in_specs=[
            pl.BlockSpec(memory_space=pltpu.SMEM),
            pl.BlockSpec((N, c), lambda r: (0, 0)),
            pl.BlockSpec((N,), lambda r: (0,)),
        ],
        out_specs=pl.BlockSpec((1, K, RT), lambda r: (r, 0, 0)),
        out_shape=jax.ShapeDtypeStruct((NT, K, RT), jnp.int32),
        scratch_shapes=[
            pltpu.VMEM((K, RT), jnp.float32),
            pltpu.VMEM((K, RT), jnp.int32),
        ],
    )(seed_flags, pos, batch)
    return out.transpose(0, 2, 1).reshape(N, K)


def _ab_body(x_ref, wt_ref, wb_ref, b_ref, a_ref, bb_ref):
    xb = x_ref[...]
    wb = wb_ref[...]
    wdiff = wt_ref[...] - wb
    a_ref[...] = (lax.dot_general(xb, wdiff, (((1,), (0,)), ((), ())),
                                  preferred_element_type=jnp.float32)
                  + b_ref[...][None, :])
    bb_ref[...] = lax.dot_general(xb, wb, (((1,), (0,)), ((), ())),
                                  preferred_element_type=jnp.float32)


def _ab(x, w1, b1):
    c = x.shape[1]
    d = w1.shape[1]
    return pl.pallas_call(
        _ab_body,
        out_shape=(jax.ShapeDtypeStruct((N, d), jnp.float32),
                   jax.ShapeDtypeStruct((N, d), jnp.float32)),
    )(x, w1[:c], w1[c:], b1)


def _sc_gather(table, idx):
    d = table.shape[1]
    if d % 128 != 0:
        pad = 128 - d % 128
        return _sc_gather(jnp.pad(table, ((0, 0), (0, pad))), idx)[:, :d]
    nw = 32
    b_per_w = NK // nw
    ch = 128
    iters = b_per_w // ch
    mesh = plsc.VectorSubcoreMesh(core_axis_name="c", subcore_axis_name="s")

    @functools.partial(
        pl.kernel, mesh=mesh,
        out_type=jax.ShapeDtypeStruct((NK, d), jnp.float32),
        scratch_types=[
            pltpu.VMEM((ch,), jnp.int32),
            pltpu.VMEM((ch, d), jnp.float32),
            pltpu.SemaphoreType.DMA,
        ],
    )
    def gk(table_hbm, idx_hbm, out_hbm, idx_v, rows_v, sem):
        wid = lax.axis_index("s") * 2 + lax.axis_index("c")
        base = wid * b_per_w

        def body(ci, carry):
            off = base + ci * ch
            pltpu.sync_copy(idx_hbm.at[pl.ds(off, ch)], idx_v)
            pltpu.async_copy(table_hbm.at[idx_v], rows_v, sem).wait()
            pltpu.sync_copy(rows_v, out_hbm.at[pl.ds(off, ch)])
            return carry

        lax.fori_loop(0, iters, body, 0)

    return gk(table, idx)


def _bcast16(a, d):
    return jnp.broadcast_to(a[:, None, :], (RT, K, d)).reshape(ET, d)


def _bn_coeffs(stats_ref, g_ref, be_ref):
    mu = stats_ref[0, :] * (1.0 / NK)
    ex2 = stats_ref[1, :] * (1.0 / NK)
    var = ex2 - mu * mu
    rstd = lax.rsqrt(var + 1e-5)
    scale = g_ref[...] * rstd
    shift = be_ref[...] - mu * scale
    return scale, shift


def _acc_stats(stats_ref, h, r):
    @pl.when(r == 0)
    def _():
        stats_ref[...] = jnp.zeros_like(stats_ref)
    stats_ref[0, :] += jnp.sum(h, axis=0)
    stats_ref[1, :] += jnp.sum(h * h, axis=0)


def _p1_body(a_ref, bj_ref, stats_ref):
    r = pl.program_id(0)
    d = a_ref.shape[1]
    h = _bcast16(a_ref[...], d) + bj_ref[...]
    _acc_stats(stats_ref, h, r)


def _p2_body(a_ref, bj_ref, st1_ref, g_ref, be_ref, w_ref, b_ref,
             e_ref, st2_ref):
    r = pl.program_id(0)
    d = a_ref.shape[1]
    scale, shift = _bn_coeffs(st1_ref, g_ref, be_ref)
    h1 = _bcast16(a_ref[...], d) + bj_ref[...]
    h1 = jnp.maximum(h1 * scale[None, :] + shift[None, :], 0.0)
    e = (lax.dot_general(h1.astype(jnp.bfloat16), w_ref[...],
                         (((1,), (0,)), ((), ())),
                         preferred_element_type=jnp.float32)
         + b_ref[...][None, :])
    e_ref[...] = e.astype(jnp.bfloat16)
    _acc_stats(st2_ref, e, r)


def _p3_body(e1_ref, st2_ref, g_ref, be_ref, w_ref, b_ref, e2_ref, st3_ref):
    r = pl.program_id(0)
    scale, shift = _bn_coeffs(st2_ref, g_ref, be_ref)
    e1 = e1_ref[...].astype(jnp.float32)
    h2 = jnp.maximum(e1 * scale[None, :] + shift[None, :], 0.0)
    e = (lax.dot_general(h2.astype(jnp.bfloat16), w_ref[...],
                         (((1,), (0,)), ((), ())),
                         preferred_element_type=jnp.float32)
         + b_ref[...][None, :])
    e2_ref[...] = e.astype(jnp.bfloat16)
    _acc_stats(st3_ref, e, r)


def _p4_body(e2_ref, st3_ref, g_ref, be_ref, h_ref):
    scale, shift = _bn_coeffs(st3_ref, g_ref, be_ref)
    d = e2_ref.shape[1]
    e2 = e2_ref[...].astype(jnp.float32)
    h3 = jnp.maximum(e2 * scale[None, :] + shift[None, :], 0.0)
    h_ref[...] = jnp.mean(h3.reshape(RT, K, d), axis=1)


def _edge_block(x, idx, w0, b0, g0, be0, w1, b1, g1, be1, w2, b2, g2, be2):
    d = w0.shape[1]
    a, b = _ab(x, w0, b0)
    bj = _sc_gather(b, idx.reshape(NK))

    espec = pl.BlockSpec((ET, d), lambda r: (r, 0))
    aspec = pl.BlockSpec((RT, d), lambda r: (r, 0))
    sspec = pl.BlockSpec((8, d), lambda r: (0, 0))
    vspec = pl.BlockSpec((d,), lambda r: (0,))
    wspec = pl.BlockSpec((d, d), lambda r: (0, 0))
    sshape = jax.ShapeDtypeStruct((8, d), jnp.float32)

    st1 = pl.pallas_call(
        _p1_body, grid=(NT,),
        in_specs=[aspec, espec],
        out_specs=sspec, out_shape=sshape,
    )(a, bj)

    e1, st2 = pl.pallas_call(
        _p2_body, grid=(NT,),
        in_specs=[aspec, espec, sspec, vspec, vspec, wspec, vspec],
        out_specs=(espec, sspec),
        out_shape=(jax.ShapeDtypeStruct((NK, d), jnp.bfloat16), sshape),
    )(a, bj, st1, g0, be0, w1.astype(jnp.bfloat16), b1)

    e2, st3 = pl.pallas_call(
        _p3_body, grid=(NT,),
        in_specs=[espec, sspec, vspec, vspec, wspec, vspec],
        out_specs=(espec, sspec),
        out_shape=(jax.ShapeDtypeStruct((NK, d), jnp.bfloat16), sshape),
    )(e1, st2, g1, be1, w2.astype(jnp.bfloat16), b2)

    h = pl.pallas_call(
        _p4_body, grid=(NT,),
        in_specs=[espec, sspec, vspec, vspec],
        out_specs=aspec,
        out_shape=jax.ShapeDtypeStruct((N, d), jnp.float32),
    )(e2, st3, g2, be2)
    return h


def _pool_body(x_ref, b_ref, w1_ref, b1_ref, w2_ref, b2_ref, out_ref,
               acc, cnt):
    r = pl.program_id(0)
    xb = x_ref[...]
    bb = b_ref[pl.ds(r * RT, RT)]
    ind = (bb[:, None] == lax.broadcasted_iota(jnp.int32, (RT, NG), 1)
           ).astype(jnp.float32)

    @pl.when(r == 0)
    def _():
        acc[...] = jnp.zeros_like(acc)
        cnt[...] = jnp.zeros_like(cnt)

    acc[...] += lax.dot_general(ind, xb, (((0,), (0,)), ((), ())),
                                preferred_element_type=jnp.float32)
    cnt[...] += jnp.sum(ind, axis=0)

    @pl.when(r == NT - 1)
    def _():
        pooled = acc[...] / jnp.maximum(cnt[...], 1.0)[:, None]
        h = (lax.dot_general(pooled, w1_ref[...], (((1,), (0,)), ((), ())),
                             preferred_element_type=jnp.float32)
             + b1_ref[...][None, :])
        h = jnp.maximum(h, 0.0)
        out_ref[...] = (lax.dot_general(h, w2_ref[...],
                                        (((1,), (0,)), ((), ())),
                                        preferred_element_type=jnp.float32)
                        + b2_ref[...][None, :])


def _pool_fc(x3, batch, wfc1, bfc1, wfc2, bfc2):
    c = x3.shape[1]
    return pl.pallas_call(
        _pool_body,
        grid=(NT,),
        in_specs=[
            pl.BlockSpec((RT, c), lambda r: (r, 0)),
            pl.BlockSpec((N,), lambda r: (0,)),
            pl.BlockSpec((c, 256), lambda r: (0, 0)),
            pl.BlockSpec((256,), lambda r: (0,)),
            pl.BlockSpec((256, 5), lambda r: (0, 0)),
            pl.BlockSpec((5,), lambda r: (0,)),
        ],
        out_specs=pl.BlockSpec((NG, 5), lambda r: (0, 0)),
        out_shape=jax.ShapeDtypeStruct((NG, 5), jnp.float32),
        scratch_shapes=[
            pltpu.VMEM((NG, c), jnp.float32),
            pltpu.VMEM((NG,), jnp.float32),
        ],
    )(x3, batch, wfc1, bfc1, wfc2, bfc2)


def kernel(x, batch,
           W0_0, b0_0, g0_0, be0_0,
           W0_1, b0_1, g0_1, be0_1,
           W0_2, b0_2, g0_2, be0_2,
           W1_0, b1_0, g1_0, be1_0,
           W1_1, b1_1, g1_1, be1_1,
           W1_2, b1_2, g1_2, be1_2,
           W2_0, b2_0, g2_0, be2_0,
           W2_1, b2_1, g2_1, be2_1,
           W2_2, b2_2, g2_2, be2_2,
           Wfc1, bfc1, Wfc2, bfc2):
    batch = batch.astype(jnp.int32)
    counts = jnp.zeros((NG,), jnp.int32).at[batch].add(1)
    b2d = batch.reshape(NT, RT)
    jets = jnp.arange(NG, dtype=jnp.int32)
    present = (jets[None, :] >= b2d[:, :1]) & (jets[None, :] <= b2d[:, -1:])
    minc = jnp.min(jnp.where(present, counts[None, :], BIG), axis=1)
    seed_flags = (minc <= K).astype(jnp.int32)
    params = [
        (W0_0, b0_0, g0_0, be0_0, W0_1, b0_1, g0_1, be0_1,
         W0_2, b0_2, g0_2, be0_2),
        (W1_0, b1_0, g1_0, be1_0, W1_1, b1_1, g1_1, be1_1,
         W1_2, b1_2, g1_2, be1_2),
        (W2_0, b2_0, g2_0, be2_0, W2_1, b2_1, g2_1, be2_1,
         W2_2, b2_2, g2_2, be2_2),
    ]
    for i in range(3):
        pos = x[:, :2] if i == 0 else x
        idx = _knn(pos, batch, seed_flags)
        h = _edge_block(x, idx, *params[i])
        x = jnp.concatenate([h, x], axis=1)
    return _pool_fc(x, batch, Wfc1, bfc1, Wfc2, bfc2)


# speedup vs baseline: 17.8514x; 1.0528x over previous
import functools

import jax
import jax.numpy as jnp
from jax import lax
from jax.experimental import pallas as pl
from jax.experimental.pallas import tpu as pltpu
from jax.experimental.pallas import tpu_sc as plsc

N = 8192
NG = 64
K = 16
NK = N * K
RT = 128
CT = 256
NCT_MAX = N // CT
ET = RT * K
NT = N // RT
INF = float("inf")
BIG = 2 ** 30


def _topk_merge(rv, ri, cv, ci):
    allv = jnp.concatenate([rv, cv], axis=0)
    alli = jnp.concatenate([ri, ci], axis=0)
    outv, outi = [], []
    for _ in range(K):
        m = jnp.min(allv, axis=0, keepdims=True)
        eq = allv == m
        sel = jnp.min(jnp.where(eq, alli, BIG), axis=0, keepdims=True)
        hit = eq & (alli == sel)
        outv.append(m)
        outi.append(sel)
        allv = jnp.where(hit, INF, allv)
        alli = jnp.where(hit, BIG, alli)
    return jnp.concatenate(outv, axis=0), jnp.concatenate(outi, axis=0)


def _knn_body(seed_ref, pos_ref, batch_ref, out_ref, rv, ri):
    r = pl.program_id(0)
    R0 = pl.multiple_of(r * RT, RT)
    rb = batch_ref[pl.ds(R0, RT)]
    full = batch_ref[...]
    b0 = jnp.min(rb)
    b1 = jnp.max(rb)
    c0 = jnp.sum((full < b0).astype(jnp.int32))
    c1 = jnp.sum((full <= b1).astype(jnp.int32))
    c0a = (c0 // 128) * 128
    nct = (c1 - c0a + CT - 1) // CT
    rowid = R0 + lax.broadcasted_iota(jnp.int32, (1, RT), 1)

    rv[...] = jnp.full((K, RT), INF, jnp.float32)
    ri[...] = jnp.full((K, RT), BIG, jnp.int32)

    @pl.when(seed_ref[r] != 0)
    def _():
        colid = lax.broadcasted_iota(jnp.int32, (CT, RT), 0)
        cb = batch_ref[pl.ds(0, CT)]
        elig = (cb[:, None] != rb[None, :]) | (colid == rowid)
        seedv = jnp.full((CT, RT), INF, jnp.float32)
        seedi = jnp.where(elig, colid, BIG)
        nv, ni = _topk_merge(rv[...], ri[...], seedv, seedi)
        rv[...] = nv
        ri[...] = ni

    rpos = pos_ref[pl.ds(R0, RT), :]
    p2r = jnp.sum(rpos * rpos, axis=1)[None, :]

    def tile_step(t, carry):
        s = c0a + t * CT
        sc = pl.multiple_of(jnp.minimum(s, N - CT), 128)
        cpos = pos_ref[pl.ds(sc, CT), :]
        cb = batch_ref[pl.ds(sc, CT)]
        colid = sc + lax.broadcasted_iota(jnp.int32, (CT, RT), 0)
        p2c = jnp.sum(cpos * cpos, axis=1)[:, None]
        dot = lax.dot_general(cpos, rpos, (((1,), (1,)), ((), ())),
                              preferred_element_type=jnp.float32)
        d2 = p2c + p2r - 2.0 * dot
        valid = ((cb[:, None] == rb[None, :]) & (colid != rowid)
                 & (colid >= s))
        cv = jnp.where(valid, d2, INF)
        ci = jnp.where(valid, colid, BIG)
        nv, ni = _topk_merge(rv[...], ri[...], cv, ci)
        rv[...] = nv
        ri[...] = ni
        return carry

    lax.fori_loop(0, nct, tile_step, 0)
    out_ref[...] = ri[...].reshape(1, K, RT)


def _knn(pos, batch, seed_flags):
    c = pos.shape[1]
    out = pl.pallas_call(
        _knn_body,
        grid=(NT,),
        in_specs=[
            pl.BlockSpec(memory_space=pltpu.SMEM),
            pl.BlockSpec((N, c), lambda r: (0, 0)),
            pl.BlockSpec((N,), lambda r: (0,)),
        ],
        out_specs=pl.BlockSpec((1, K, RT), lambda r: (r, 0, 0)),
        out_shape=jax.ShapeDtypeStruct((NT, K, RT), jnp.int32),
        scratch_shapes=[
            pltpu.VMEM((K, RT), jnp.float32),
            pltpu.VMEM((K, RT), jnp.int32),
        ],
    )(seed_flags, pos, batch)
    return out.transpose(0, 2, 1).reshape(N, K)


def _ab_body(x_ref, wt_ref, wb_ref, b_ref, a_ref, bb_ref):
    xb = x_ref[...]
    wb = wb_ref[...]
    wdiff = wt_ref[...] - wb
    a_ref[...] = (lax.dot_general(xb, wdiff, (((1,), (0,)), ((), ())),
                                  preferred_element_type=jnp.float32)
                  + b_ref[...][None, :])
    bb_ref[...] = lax.dot_general(xb, wb, (((1,), (0,)), ((), ())),
                                  preferred_element_type=jnp.float32)


def _ab(x, w1, b1):
    c = x.shape[1]
    d = w1.shape[1]
    return pl.pallas_call(
        _ab_body,
        out_shape=(jax.ShapeDtypeStruct((N, d), jnp.float32),
                   jax.ShapeDtypeStruct((N, d), jnp.float32)),
    )(x, w1[:c], w1[c:], b1)


def _sc_gather(table, idx):
    d = table.shape[1]
    if d % 128 != 0:
        pad = 128 - d % 128
        return _sc_gather(jnp.pad(table, ((0, 0), (0, pad))), idx)[:, :d]
    nw = 32
    b_per_w = NK // nw
    ch = 128
    pairs = b_per_w // (2 * ch)
    mesh = plsc.VectorSubcoreMesh(core_axis_name="c", subcore_axis_name="s")

    @functools.partial(
        pl.kernel, mesh=mesh,
        out_type=jax.ShapeDtypeStruct((NK, d), jnp.float32),
        scratch_types=[
            pltpu.VMEM((b_per_w,), jnp.int32),
            pltpu.VMEM((ch, d), jnp.float32),
            pltpu.VMEM((ch, d), jnp.float32),
            pltpu.SemaphoreType.DMA,
            pltpu.SemaphoreType.DMA,
            pltpu.SemaphoreType.DMA,
            pltpu.SemaphoreType.DMA,
        ],
    )
    def gk(table_hbm, idx_hbm, out_hbm, idx_v, rows0, rows1, g0, g1,
           w0, w1):
        wid = lax.axis_index("s") * 2 + lax.axis_index("c")
        base = wid * b_per_w
        pltpu.sync_copy(idx_hbm.at[pl.ds(base, b_per_w)], idx_v)

        def body(pi, carry):
            c0 = 2 * pi * ch
            c1 = c0 + ch
            wb0 = pltpu.make_async_copy(
                rows0, out_hbm.at[pl.ds(base + c0, ch)], w0)
            wb1 = pltpu.make_async_copy(
                rows1, out_hbm.at[pl.ds(base + c1, ch)], w1)

            @pl.when(pi > 0)
            def _():
                wb0.wait()
                wb1.wait()

            ga0 = pltpu.make_async_copy(
                table_hbm.at[idx_v.at[pl.ds(c0, ch)]], rows0, g0)
            ga1 = pltpu.make_async_copy(
                table_hbm.at[idx_v.at[pl.ds(c1, ch)]], rows1, g1)
            ga0.start()
            ga1.start()
            ga0.wait()
            wb0.start()
            ga1.wait()
            wb1.start()
            return carry

        lax.fori_loop(0, pairs, body, 0)
        pltpu.make_async_copy(
            rows0, out_hbm.at[pl.ds(base, ch)], w0).wait()
        pltpu.make_async_copy(
            rows1, out_hbm.at[pl.ds(base, ch)], w1).wait()

    return gk(table, idx)


def _bcast16(a, d):
    return jnp.broadcast_to(a[:, None, :], (RT, K, d)).reshape(ET, d)


def _bn_coeffs(stats_ref, g_ref, be_ref):
    mu = stats_ref[0, :] * (1.0 / NK)
    ex2 = stats_ref[1, :] * (1.0 / NK)
    var = ex2 - mu * mu
    rstd = lax.rsqrt(var + 1e-5)
    scale = g_ref[...] * rstd
    shift = be_ref[...] - mu * scale
    return scale, shift


def _acc_stats(stats_ref, h, r):
    @pl.when(r == 0)
    def _():
        stats_ref[...] = jnp.zeros_like(stats_ref)
    stats_ref[0, :] += jnp.sum(h, axis=0)
    stats_ref[1, :] += jnp.sum(h * h, axis=0)


def _p1_body(a_ref, bj_ref, stats_ref):
    r = pl.program_id(0)
    d = a_ref.shape[1]
    h = _bcast16(a_ref[...], d) + bj_ref[...]
    _acc_stats(stats_ref, h, r)


def _p2_body(a_ref, bj_ref, st1_ref, g_ref, be_ref, w_ref, b_ref,
             e_ref, st2_ref):
    r = pl.program_id(0)
    d = a_ref.shape[1]
    scale, shift = _bn_coeffs(st1_ref, g_ref, be_ref)
    h1 = _bcast16(a_ref[...], d) + bj_ref[...]
    h1 = jnp.maximum(h1 * scale[None, :] + shift[None, :], 0.0)
    e = (lax.dot_general(h1.astype(jnp.bfloat16), w_ref[...],
                         (((1,), (0,)), ((), ())),
                         preferred_element_type=jnp.float32)
         + b_ref[...][None, :])
    e_ref[...] = e.astype(jnp.bfloat16)
    _acc_stats(st2_ref, e, r)


def _p3_body(e1_ref, st2_ref, g_ref, be_ref, w_ref, b_ref, e2_ref, st3_ref):
    r = pl.program_id(0)
    scale, shift = _bn_coeffs(st2_ref, g_ref, be_ref)
    e1 = e1_ref[...].astype(jnp.float32)
    h2 = jnp.maximum(e1 * scale[None, :] + shift[None, :], 0.0)
    e = (lax.dot_general(h2.astype(jnp.bfloat16), w_ref[...],
                         (((1,), (0,)), ((), ())),
                         preferred_element_type=jnp.float32)
         + b_ref[...][None, :])
    e2_ref[...] = e.astype(jnp.bfloat16)
    _acc_stats(st3_ref, e, r)


def _p4_body(e2_ref, st3_ref, g_ref, be_ref, h_ref):
    scale, shift = _bn_coeffs(st3_ref, g_ref, be_ref)
    d = e2_ref.shape[1]
    e2 = e2_ref[...].astype(jnp.float32)
    h3 = jnp.maximum(e2 * scale[None, :] + shift[None, :], 0.0)
    h_ref[...] = jnp.mean(h3.reshape(RT, K, d), axis=1)


def _edge_block(x, idx, w0, b0, g0, be0, w1, b1, g1, be1, w2, b2, g2, be2):
    d = w0.shape[1]
    a, b = _ab(x, w0, b0)
    bj = _sc_gather(b, idx.reshape(NK))

    espec = pl.BlockSpec((ET, d), lambda r: (r, 0))
    aspec = pl.BlockSpec((RT, d), lambda r: (r, 0))
    sspec = pl.BlockSpec((8, d), lambda r: (0, 0))
    vspec = pl.BlockSpec((d,), lambda r: (0,))
    wspec = pl.BlockSpec((d, d), lambda r: (0, 0))
    sshape = jax.ShapeDtypeStruct((8, d), jnp.float32)

    st1 = pl.pallas_call(
        _p1_body, grid=(NT,),
        in_specs=[aspec, espec],
        out_specs=sspec, out_shape=sshape,
    )(a, bj)

    e1, st2 = pl.pallas_call(
        _p2_body, grid=(NT,),
        in_specs=[aspec, espec, sspec, vspec, vspec, wspec, vspec],
        out_specs=(espec, sspec),
        out_shape=(jax.ShapeDtypeStruct((NK, d), jnp.bfloat16), sshape),
    )(a, bj, st1, g0, be0, w1.astype(jnp.bfloat16), b1)

    e2, st3 = pl.pallas_call(
        _p3_body, grid=(NT,),
        in_specs=[espec, sspec, vspec, vspec, wspec, vspec],
        out_specs=(espec, sspec),
        out_shape=(jax.ShapeDtypeStruct((NK, d), jnp.bfloat16), sshape),
    )(e1, st2, g1, be1, w2.astype(jnp.bfloat16), b2)

    h = pl.pallas_call(
        _p4_body, grid=(NT,),
        in_specs=[espec, sspec, vspec, vspec],
        out_specs=aspec,
        out_shape=jax.ShapeDtypeStruct((N, d), jnp.float32),
    )(e2, st3, g2, be2)
    return h


def _pool_body(x_ref, b_ref, w1_ref, b1_ref, w2_ref, b2_ref, out_ref,
               acc, cnt):
    r = pl.program_id(0)
    xb = x_ref[...]
    bb = b_ref[pl.ds(r * RT, RT)]
    ind = (bb[:, None] == lax.broadcasted_iota(jnp.int32, (RT, NG), 1)
           ).astype(jnp.float32)

    @pl.when(r == 0)
    def _():
        acc[...] = jnp.zeros_like(acc)
        cnt[...] = jnp.zeros_like(cnt)

    acc[...] += lax.dot_general(ind, xb, (((0,), (0,)), ((), ())),
                                preferred_element_type=jnp.float32)
    cnt[...] += jnp.sum(ind, axis=0)

    @pl.when(r == NT - 1)
    def _():
        pooled = acc[...] / jnp.maximum(cnt[...], 1.0)[:, None]
        h = (lax.dot_general(pooled, w1_ref[...], (((1,), (0,)), ((), ())),
                             preferred_element_type=jnp.float32)
             + b1_ref[...][None, :])
        h = jnp.maximum(h, 0.0)
        out_ref[...] = (lax.dot_general(h, w2_ref[...],
                                        (((1,), (0,)), ((), ())),
                                        preferred_element_type=jnp.float32)
                        + b2_ref[...][None, :])


def _pool_fc(x3, batch, wfc1, bfc1, wfc2, bfc2):
    c = x3.shape[1]
    return pl.pallas_call(
        _pool_body,
        grid=(NT,),
        in_specs=[
            pl.BlockSpec((RT, c), lambda r: (r, 0)),
            pl.BlockSpec((N,), lambda r: (0,)),
            pl.BlockSpec((c, 256), lambda r: (0, 0)),
            pl.BlockSpec((256,), lambda r: (0,)),
            pl.BlockSpec((256, 5), lambda r: (0, 0)),
            pl.BlockSpec((5,), lambda r: (0,)),
        ],
        out_specs=pl.BlockSpec((NG, 5), lambda r: (0, 0)),
        out_shape=jax.ShapeDtypeStruct((NG, 5), jnp.float32),
        scratch_shapes=[
            pltpu.VMEM((NG, c), jnp.float32),
            pltpu.VMEM((NG,), jnp.float32),
        ],
    )(x3, batch, wfc1, bfc1, wfc2, bfc2)


def kernel(x, batch,
           W0_0, b0_0, g0_0, be0_0,
           W0_1, b0_1, g0_1, be0_1,
           W0_2, b0_2, g0_2, be0_2,
           W1_0, b1_0, g1_0, be1_0,
           W1_1, b1_1, g1_1, be1_1,
           W1_2, b1_2, g1_2, be1_2,
           W2_0, b2_0, g2_0, be2_0,
           W2_1, b2_1, g2_1, be2_1,
           W2_2, b2_2, g2_2, be2_2,
           Wfc1, bfc1, Wfc2, bfc2):
    batch = batch.astype(jnp.int32)
    counts = jnp.zeros((NG,), jnp.int32).at[batch].add(1)
    b2d = batch.reshape(NT, RT)
    jets = jnp.arange(NG, dtype=jnp.int32)
    present = (jets[None, :] >= b2d[:, :1]) & (jets[None, :] <= b2d[:, -1:])
    minc = jnp.min(jnp.where(present, counts[None, :], BIG), axis=1)
    seed_flags = (minc <= K).astype(jnp.int32)
    params = [
        (W0_0, b0_0, g0_0, be0_0, W0_1, b0_1, g0_1, be0_1,
         W0_2, b0_2, g0_2, be0_2),
        (W1_0, b1_0, g1_0, be1_0, W1_1, b1_1, g1_1, be1_1,
         W1_2, b1_2, g1_2, be1_2),
        (W2_0, b2_0, g2_0, be2_0, W2_1, b2_1, g2_1, be2_1,
         W2_2, b2_2, g2_2, be2_2),
    ]
    for i in range(3):
        pos = x[:, :2] if i == 0 else x
        idx = _knn(pos, batch, seed_flags)
        h = _edge_block(x, idx, *params[i])
        x = jnp.concatenate([h, x], axis=1)
    return _pool_fc(x, batch, Wfc1, bfc1, Wfc2, bfc2)
